# baseline (speedup 1.0000x reference)
import functools
import math

import jax
import jax.numpy as jnp
import numpy as np
from jax import lax
from jax.experimental import pallas as pl
from jax.experimental.pallas import tpu as pltpu
from jax.experimental.pallas import tpu_sc as plsc

_SR = 0.1
_B = 128
_N = 32768
_K = int(math.ceil(_SR * _N))
_L = 16
_NC = 2
_NS = 16
_NW = _NC * _NS
_ROWS_PER_W = _B // _NW
_NV = _N // _L
_UNROLL = 8

_MASK31 = np.int32(0x7FFFFFFF)
_INT_MIN = np.int32(-2147483648)


def _lane_permute(v, idx):
    return lax.gather(
        v, idx[:, None],
        dimension_numbers=lax.GatherDimensionNumbers(
            offset_dims=(), collapsed_slice_dims=(0,), start_index_map=(0,)),
        slice_sizes=(1,),
        mode=lax.GatherScatterMode.PROMISE_IN_BOUNDS)


def _lane_sum(v):
    lane = lax.iota(jnp.int32, _L)
    for sh in (8, 4, 2, 1):
        v = v + _lane_permute(v, lane ^ sh)
    return v


def _sc_sparsify(x_hbm, o_hbm, xbuf, kbuf):
    wid = lax.axis_index("s") * _NC + lax.axis_index("c")
    kvec = jnp.full((_L,), jnp.float32(_K))
    for r in range(_ROWS_PER_W):
        row = wid * _ROWS_PER_W + r
        base = row * _N
        pltpu.sync_copy(x_hbm.at[pl.ds(base, _N)], xbuf)

        def stage(jj, carry):
            for u in range(_UNROLL):
                sl = pl.ds((jj * _UNROLL + u) * _L, _L)
                b = lax.bitcast_convert_type(xbuf[sl], jnp.int32)
                kbuf[sl] = jnp.where(b >= 0, b, b ^ _MASK31)
            return carry
        lax.fori_loop(0, _NV // _UNROLL, stage, np.int32(0))

        def outer(i, t):
            bit = lax.shift_left(
                jnp.full((_L,), np.int32(1)),
                jnp.full((_L,), np.int32(31)) - i)
            cand = t + bit

            def cnt_body(jj, cvec):
                for u in range(_UNROLL):
                    sl = pl.ds((jj * _UNROLL + u) * _L, _L)
                    cvec = cvec + jnp.where(kbuf[sl] >= cand,
                                            jnp.float32(1.0), jnp.float32(0.0))
                return cvec

            cvec = lax.fori_loop(0, _NV // _UNROLL, cnt_body,
                                 jnp.zeros((_L,), jnp.float32))
            total = _lane_sum(cvec)
            return jnp.where(total >= kvec, cand, t)

        t = lax.fori_loop(0, 32, outer, jnp.full((_L,), _INT_MIN))

        def mask_body(jj, svec):
            for u in range(_UNROLL):
                sl = pl.ds((jj * _UNROLL + u) * _L, _L)
                res = jnp.where(kbuf[sl] >= t, xbuf[sl], jnp.float32(0.0))
                xbuf[sl] = res
                svec = svec + res
            return svec

        svec = lax.fori_loop(0, _NV // _UNROLL, mask_body,
                             jnp.zeros((_L,), jnp.float32))
        rowsum = _lane_sum(svec)
        scale_vec = jnp.full((_L,), jnp.float32(_N)) / rowsum

        def scale_body(jj, carry):
            for u in range(_UNROLL):
                sl = pl.ds((jj * _UNROLL + u) * _L, _L)
                xbuf[sl] = xbuf[sl] * scale_vec
            return carry
        lax.fori_loop(0, _NV // _UNROLL, scale_body, np.int32(0))

        pltpu.sync_copy(xbuf, o_hbm.at[pl.ds(base, _N)])


_sc_call = functools.partial(
    pl.kernel,
    mesh=plsc.VectorSubcoreMesh(core_axis_name="c", subcore_axis_name="s"),
    out_type=jax.ShapeDtypeStruct((_B * _N,), jnp.float32),
    scratch_types=[
        pltpu.VMEM((_N,), jnp.float32),
        pltpu.VMEM((_N,), jnp.int32),
    ],
)(_sc_sparsify)


def kernel(x):
    y = _sc_call(x.reshape(-1))
    return y.reshape(_B, _N)


# speedup vs baseline: 2.0538x; 2.0538x over previous
import functools
import math

import jax
import jax.numpy as jnp
import numpy as np
from jax import lax
from jax.experimental import pallas as pl
from jax.experimental.pallas import tpu as pltpu
from jax.experimental.pallas import tpu_sc as plsc

_SR = 0.1
_B = 128
_N = 32768
_K = int(math.ceil(_SR * _N))
_L = 16
_NC = 2
_NS = 16
_NW = _NC * _NS
_NV = _N // _L
_UNROLL = 8

_SC_ROWS = 32

_MASK31 = np.int32(0x7FFFFFFF)
_INT_MIN = np.int32(-2147483648)


def _sc_sparsify(rows_per_w, x_hbm, o_hbm, xbuf, kbuf):
    wid = lax.axis_index("s") * _NC + lax.axis_index("c")
    kvec = jnp.full((_L,), np.int32(_K))
    for r in range(rows_per_w):
        base = (wid * rows_per_w + r) * _N
        pltpu.sync_copy(x_hbm.at[pl.ds(base, _N)], xbuf)

        def stage(jj, carry):
            for u in range(_UNROLL):
                sl = pl.ds((jj * _UNROLL + u) * _L, _L)
                b = lax.bitcast_convert_type(xbuf[sl], jnp.int32)
                kbuf[sl] = jnp.where(b >= 0, b, b ^ _MASK31)
            return carry
        lax.fori_loop(0, _NV // _UNROLL, stage, np.int32(0))

        def outer(i, t):
            bit = lax.shift_left(
                jnp.full((_L,), np.int32(1)),
                jnp.full((_L,), np.int32(31)) - i)
            cand = t + bit

            def cnt_body(jj, cvec):
                for u in range(_UNROLL):
                    sl = pl.ds((jj * _UNROLL + u) * _L, _L)
                    cvec = cvec + jnp.where(kbuf[sl] >= cand, 1, 0)
                return cvec

            cvec = lax.fori_loop(0, _NV // _UNROLL, cnt_body,
                                 jnp.zeros((_L,), jnp.int32))
            total = _lane_sum(cvec)
            return jnp.where(total >= kvec, cand, t)

        t = lax.fori_loop(0, 32, outer, jnp.full((_L,), _INT_MIN))

        def mask_body(jj, svec):
            for u in range(_UNROLL):
                sl = pl.ds((jj * _UNROLL + u) * _L, _L)
                res = jnp.where(kbuf[sl] >= t, xbuf[sl], jnp.float32(0.0))
                xbuf[sl] = res
                svec = svec + res
            return svec

        svec = lax.fori_loop(0, _NV // _UNROLL, mask_body,
                             jnp.zeros((_L,), jnp.float32))
        rowsum = _lane_sum(svec)
        scale_vec = jnp.full((_L,), jnp.float32(_N)) / rowsum

        def scale_body(jj, carry):
            for u in range(_UNROLL):
                sl = pl.ds((jj * _UNROLL + u) * _L, _L)
                xbuf[sl] = xbuf[sl] * scale_vec
            return carry
        lax.fori_loop(0, _NV // _UNROLL, scale_body, np.int32(0))

        pltpu.sync_copy(xbuf, o_hbm.at[pl.ds(base, _N)])


def _lane_permute(v, idx):
    return lax.gather(
        v, idx[:, None],
        dimension_numbers=lax.GatherDimensionNumbers(
            offset_dims=(), collapsed_slice_dims=(0,), start_index_map=(0,)),
        slice_sizes=(1,),
        mode=lax.GatherScatterMode.PROMISE_IN_BOUNDS)


def _lane_sum(v):
    lane = lax.iota(jnp.int32, _L)
    for sh in (8, 4, 2, 1):
        v = v + _lane_permute(v, lane ^ sh)
    return v


def _make_sc_call(nrows):
    rows_per_w = nrows // _NW
    assert rows_per_w * _NW == nrows
    return functools.partial(
        pl.kernel,
        mesh=plsc.VectorSubcoreMesh(core_axis_name="c", subcore_axis_name="s"),
        out_type=jax.ShapeDtypeStruct((nrows * _N,), jnp.float32),
        scratch_types=[
            pltpu.VMEM((_N,), jnp.float32),
            pltpu.VMEM((_N,), jnp.int32),
        ],
    )(functools.partial(_sc_sparsify, rows_per_w))


def _tc_block(x_ref, o_ref):
    x = x_ref[...]
    bits = lax.bitcast_convert_type(x, jnp.int32)
    s = jnp.where(bits >= 0, bits, bits ^ _MASK31)

    def body(i, t):
        bit = lax.shift_left(jnp.int32(1), jnp.int32(31) - i)
        cand = t + bit
        cnt = jnp.sum((s >= cand).astype(jnp.int32), axis=-1, keepdims=True)
        return jnp.where(cnt >= _K, cand, t)

    t0 = jnp.full((x.shape[0], 1), _INT_MIN)
    t = lax.fori_loop(0, 32, body, t0)
    tb = jnp.where(t >= 0, t, t ^ _MASK31)
    thr = lax.bitcast_convert_type(tb, jnp.float32)
    res = jnp.where(x >= thr, x, jnp.float32(0.0))
    denom = jnp.sum(res, axis=-1, keepdims=True) / jnp.float32(_N)
    o_ref[...] = res / denom


def _tc_call(x):
    b = x.shape[0]
    rows = 8
    return pl.pallas_call(
        _tc_block,
        grid=(b // rows,),
        in_specs=[pl.BlockSpec((rows, _N), lambda i: (i, 0))],
        out_specs=pl.BlockSpec((rows, _N), lambda i: (i, 0)),
        out_shape=jax.ShapeDtypeStruct((b, _N), jnp.float32),
    )(x)


def kernel(x):
    sc_call = _make_sc_call(_SC_ROWS)
    ys = sc_call(x[:_SC_ROWS].reshape(-1)).reshape(_SC_ROWS, _N)
    yt = _tc_call(x[_SC_ROWS:])
    return jnp.concatenate([ys, yt], axis=0)


# speedup vs baseline: 2.6364x; 1.2837x over previous
import functools
import math

import jax
import jax.numpy as jnp
import numpy as np
from jax import lax
from jax.experimental import pallas as pl
from jax.experimental.pallas import tpu as pltpu
from jax.experimental.pallas import tpu_sc as plsc

_SR = 0.1
_B = 128
_N = 32768
_K = int(math.ceil(_SR * _N))
_L = 16
_NC = 2
_NS = 16
_NW = _NC * _NS
_NV = _N // _L
_UNROLL = 8

_SC_ROWS = 64

_MASK31 = np.int32(0x7FFFFFFF)
_INT_MIN = np.int32(-2147483648)


def _sc_sparsify(rows_per_w, x_hbm, o_hbm, xbuf, kbuf):
    wid = lax.axis_index("s") * _NC + lax.axis_index("c")
    kvec = jnp.full((_L,), np.int32(_K))
    for r in range(rows_per_w):
        base = (wid * rows_per_w + r) * _N
        pltpu.sync_copy(x_hbm.at[pl.ds(base, _N)], xbuf)

        def stage(jj, carry):
            for u in range(_UNROLL):
                sl = pl.ds((jj * _UNROLL + u) * _L, _L)
                b = lax.bitcast_convert_type(xbuf[sl], jnp.int32)
                kbuf[sl] = jnp.where(b >= 0, b, b ^ _MASK31)
            return carry
        lax.fori_loop(0, _NV // _UNROLL, stage, np.int32(0))

        def outer(i, t):
            bit = lax.shift_left(
                jnp.full((_L,), np.int32(1)),
                jnp.full((_L,), np.int32(31)) - i)
            cand = t + bit

            def cnt_body(jj, cvec):
                for u in range(_UNROLL):
                    sl = pl.ds((jj * _UNROLL + u) * _L, _L)
                    cvec = cvec + jnp.where(kbuf[sl] >= cand, 1, 0)
                return cvec

            cvec = lax.fori_loop(0, _NV // _UNROLL, cnt_body,
                                 jnp.zeros((_L,), jnp.int32))
            total = _lane_sum(cvec)
            return jnp.where(total >= kvec, cand, t)

        t = lax.fori_loop(0, 32, outer, jnp.full((_L,), _INT_MIN))

        def mask_body(jj, svec):
            for u in range(_UNROLL):
                sl = pl.ds((jj * _UNROLL + u) * _L, _L)
                res = jnp.where(kbuf[sl] >= t, xbuf[sl], jnp.float32(0.0))
                xbuf[sl] = res
                svec = svec + res
            return svec

        svec = lax.fori_loop(0, _NV // _UNROLL, mask_body,
                             jnp.zeros((_L,), jnp.float32))
        rowsum = _lane_sum(svec)
        scale_vec = jnp.full((_L,), jnp.float32(_N)) / rowsum

        def scale_body(jj, carry):
            for u in range(_UNROLL):
                sl = pl.ds((jj * _UNROLL + u) * _L, _L)
                xbuf[sl] = xbuf[sl] * scale_vec
            return carry
        lax.fori_loop(0, _NV // _UNROLL, scale_body, np.int32(0))

        pltpu.sync_copy(xbuf, o_hbm.at[pl.ds(base, _N)])


def _lane_permute(v, idx):
    return lax.gather(
        v, idx[:, None],
        dimension_numbers=lax.GatherDimensionNumbers(
            offset_dims=(), collapsed_slice_dims=(0,), start_index_map=(0,)),
        slice_sizes=(1,),
        mode=lax.GatherScatterMode.PROMISE_IN_BOUNDS)


def _lane_sum(v):
    lane = lax.iota(jnp.int32, _L)
    for sh in (8, 4, 2, 1):
        v = v + _lane_permute(v, lane ^ sh)
    return v


def _make_sc_call(nrows):
    rows_per_w = nrows // _NW
    assert rows_per_w * _NW == nrows
    return functools.partial(
        pl.kernel,
        mesh=plsc.VectorSubcoreMesh(core_axis_name="c", subcore_axis_name="s"),
        out_type=jax.ShapeDtypeStruct((nrows * _N,), jnp.float32),
        scratch_types=[
            pltpu.VMEM((_N,), jnp.float32),
            pltpu.VMEM((_N,), jnp.int32),
        ],
    )(functools.partial(_sc_sparsify, rows_per_w))


def _tc_block(x_ref, o_ref):
    x = x_ref[...]
    bits = lax.bitcast_convert_type(x, jnp.int32)
    s = jnp.where(bits >= 0, bits, bits ^ _MASK31)

    def body(i, t):
        bit = lax.shift_left(jnp.int32(1), jnp.int32(31) - i)
        cand = t + bit
        cnt = jnp.sum((s >= cand).astype(jnp.int32), axis=-1, keepdims=True)
        return jnp.where(cnt >= _K, cand, t)

    t0 = jnp.full((x.shape[0], 1), _INT_MIN)
    t = lax.fori_loop(0, 32, body, t0)
    tb = jnp.where(t >= 0, t, t ^ _MASK31)
    thr = lax.bitcast_convert_type(tb, jnp.float32)
    res = jnp.where(x >= thr, x, jnp.float32(0.0))
    denom = jnp.sum(res, axis=-1, keepdims=True) / jnp.float32(_N)
    o_ref[...] = res / denom


def _tc_call(x):
    b = x.shape[0]
    rows = 8
    return pl.pallas_call(
        _tc_block,
        grid=(b // rows,),
        in_specs=[pl.BlockSpec((rows, _N), lambda i: (i, 0))],
        out_specs=pl.BlockSpec((rows, _N), lambda i: (i, 0)),
        out_shape=jax.ShapeDtypeStruct((b, _N), jnp.float32),
    )(x)


def kernel(x):
    sc_call = _make_sc_call(_SC_ROWS)
    ys = sc_call(x[:_SC_ROWS].reshape(-1)).reshape(_SC_ROWS, _N)
    yt = _tc_call(x[_SC_ROWS:])
    return jnp.concatenate([ys, yt], axis=0)


# speedup vs baseline: 3.3861x; 1.2844x over previous
import functools
import math

import jax
import jax.numpy as jnp
import numpy as np
from jax import lax
from jax.experimental import pallas as pl
from jax.experimental.pallas import tpu as pltpu
from jax.experimental.pallas import tpu_sc as plsc

_SR = 0.1
_B = 128
_N = 32768
_K = int(math.ceil(_SR * _N))
_L = 16
_NC = 2
_NS = 16
_NW = _NC * _NS
_NV = _N // _L
_UNROLL = 8

_SC_ROWS = 64

_MASK31 = np.int32(0x7FFFFFFF)
_INT_MIN = np.int32(-2147483648)


def _sc_sparsify(rows_per_w, x_hbm, o_hbm, xbuf0, xbuf1, kbuf,
                 insem0, insem1, outsem):
    wid = lax.axis_index("s") * _NC + lax.axis_index("c")
    kvec = jnp.full((_L,), np.int32(_K))
    xbufs = (xbuf0, xbuf1)
    insems = (insem0, insem1)
    row0 = wid * rows_per_w
    in_cp = [None] * rows_per_w
    out_cp = []
    in_cp[0] = pltpu.async_copy(
        x_hbm.at[pl.ds(row0, 1), :], xbufs[0], insems[0])
    for r in range(rows_per_w):
        row = row0 + r
        xbuf = xbufs[r % 2]
        in_cp[r].wait()
        if r + 1 < rows_per_w:
            in_cp[r + 1] = pltpu.async_copy(
                x_hbm.at[pl.ds(row + 1, 1), :],
                xbufs[(r + 1) % 2], insems[(r + 1) % 2])

        def stage(jj, carry):
            for u in range(_UNROLL):
                sl = (0, pl.ds((jj * _UNROLL + u) * _L, _L))
                b = lax.bitcast_convert_type(xbuf[sl], jnp.int32)
                kbuf[sl] = jnp.where(b >= 0, b, b ^ _MASK31)
            return carry
        lax.fori_loop(0, _NV // _UNROLL, stage, np.int32(0))

        def outer(i, t):
            bit = lax.shift_left(
                jnp.full((_L,), np.int32(1)),
                jnp.full((_L,), np.int32(31)) - i)
            cand = t + bit

            def cnt_body(jj, cvec):
                for u in range(_UNROLL):
                    sl = (0, pl.ds((jj * _UNROLL + u) * _L, _L))
                    cvec = cvec + jnp.where(kbuf[sl] >= cand, 1, 0)
                return cvec

            cvec = lax.fori_loop(0, _NV // _UNROLL, cnt_body,
                                 jnp.zeros((_L,), jnp.int32))
            total = _lane_sum(cvec)
            return jnp.where(total >= kvec, cand, t)

        t = lax.fori_loop(0, 32, outer, jnp.full((_L,), _INT_MIN))

        def mask_body(jj, svec):
            for u in range(_UNROLL):
                sl = (0, pl.ds((jj * _UNROLL + u) * _L, _L))
                res = jnp.where(kbuf[sl] >= t, xbuf[sl], jnp.float32(0.0))
                xbuf[sl] = res
                svec = svec + res
            return svec

        svec = lax.fori_loop(0, _NV // _UNROLL, mask_body,
                             jnp.zeros((_L,), jnp.float32))
        rowsum = _lane_sum(svec)
        scale_vec = jnp.full((_L,), jnp.float32(_N)) / rowsum

        def scale_body(jj, carry):
            for u in range(_UNROLL):
                sl = (0, pl.ds((jj * _UNROLL + u) * _L, _L))
                xbuf[sl] = xbuf[sl] * scale_vec
            return carry
        lax.fori_loop(0, _NV // _UNROLL, scale_body, np.int32(0))

        out_cp.append(pltpu.async_copy(
            xbuf, o_hbm.at[pl.ds(row, 1), :], outsem))
    for cp in out_cp:
        cp.wait()


def _lane_permute(v, idx):
    return lax.gather(
        v, idx[:, None],
        dimension_numbers=lax.GatherDimensionNumbers(
            offset_dims=(), collapsed_slice_dims=(0,), start_index_map=(0,)),
        slice_sizes=(1,),
        mode=lax.GatherScatterMode.PROMISE_IN_BOUNDS)


def _lane_sum(v):
    lane = lax.iota(jnp.int32, _L)
    for sh in (8, 4, 2, 1):
        v = v + _lane_permute(v, lane ^ sh)
    return v


def _make_sc_call(nrows):
    rows_per_w = nrows // _NW
    assert rows_per_w * _NW == nrows
    return functools.partial(
        pl.kernel,
        mesh=plsc.VectorSubcoreMesh(core_axis_name="c", subcore_axis_name="s"),
        out_type=jax.ShapeDtypeStruct((_B, _N), jnp.float32),
        scratch_types=[
            pltpu.VMEM((1, _N), jnp.float32),
            pltpu.VMEM((1, _N), jnp.float32),
            pltpu.VMEM((1, _N), jnp.int32),
            pltpu.SemaphoreType.DMA,
            pltpu.SemaphoreType.DMA,
            pltpu.SemaphoreType.DMA,
        ],
    )(functools.partial(_sc_sparsify, rows_per_w))


def _tc_block(x_ref, o_ref):
    x = x_ref[...]
    rows = x.shape[0]
    bits = lax.bitcast_convert_type(x, jnp.int32)
    s = jnp.where(bits >= 0, bits, bits ^ _MASK31)

    s16 = lax.shift_right_arithmetic(s, 16).astype(jnp.int16)

    def body_hi(i, t):
        bit = lax.shift_left(jnp.int32(1), jnp.int32(15) - i)
        cand = t + bit
        comp = (s16 >= cand.astype(jnp.int16)).astype(jnp.int16)
        w = _N
        while w > 512:
            w //= 2
            comp = comp[:, :w] + comp[:, w:]
        cnt = jnp.sum(comp.astype(jnp.int32), axis=-1, keepdims=True)
        return jnp.where(cnt >= _K, cand, t)

    t16 = lax.fori_loop(0, 16, body_hi,
                        jnp.full((rows, 1), np.int32(-32768)))

    def body_lo(i, t):
        bit = lax.shift_left(jnp.int32(1), jnp.int32(15) - i)
        cand = t + bit
        cnt = jnp.sum((s >= cand).astype(jnp.int32), axis=-1, keepdims=True)
        return jnp.where(cnt >= _K, cand, t)

    t = lax.fori_loop(0, 16, body_lo, lax.shift_left(t16, 16))
    tb = jnp.where(t >= 0, t, t ^ _MASK31)
    thr = lax.bitcast_convert_type(tb, jnp.float32)
    res = jnp.where(x >= thr, x, jnp.float32(0.0))
    denom = jnp.sum(res, axis=-1, keepdims=True) / jnp.float32(_N)
    o_ref[...] = res / denom


def _tc_call(x, row0):
    b = x.shape[0] - row0
    rows = 8
    blk0 = row0 // rows
    return pl.pallas_call(
        _tc_block,
        grid=(b // rows,),
        in_specs=[pl.BlockSpec((rows, _N), lambda i: (i + blk0, 0))],
        out_specs=pl.BlockSpec((rows, _N), lambda i: (i, 0)),
        out_shape=jax.ShapeDtypeStruct((b, _N), jnp.float32),
    )(x)


def kernel(x):
    sc_call = _make_sc_call(_SC_ROWS)
    ys = sc_call(x)
    yt = _tc_call(x, _SC_ROWS)
    return lax.dynamic_update_slice(ys, yt, (_SC_ROWS, 0))
